# Initial kernel scaffold; baseline (speedup 1.0000x reference)
import jax
import jax.numpy as jnp
from jax.experimental import pallas as pl


def kernel(x, pos, edge_index, edge_attr, batch, W_in, b_in, mu, We1, be1, W1, b1, We2, be2, W2, b2, We3, be3, W3, b3):
    raise NotImplementedError("write your pallas kernel here")


# speedup vs baseline: 3.1532x; 3.1532x over previous
import dataclasses
import functools

import jax
import jax.numpy as jnp
from jax import lax
from jax.experimental import pallas as pl
from jax.experimental.pallas import tpu as pltpu
from jax.experimental.pallas import tpu_sc as plsc

_N = 10000
_E = 320000
_D = 128
_NRBF = 32
_NG = 32
_GAMMA = 10.0

_NC = 2
_NS = 16
_NTILES = _NC * _NS
_EPT = _E // _NTILES
_KC = 80
_NCHUNK = _EPT // _KC
_RSTRIPE = 632
_RLAST = _N - 15 * _RSTRIPE
_L = 16

_mesh = plsc.VectorSubcoreMesh(core_axis_name="c", subcore_axis_name="s")

_sc_params = pltpu.CompilerParams()
if "needs_layout_passes" in pltpu.CompilerParams.__dataclass_fields__:
    _sc_params = dataclasses.replace(_sc_params, needs_layout_passes=False)


def _d2_body(row_hbm, col_hbm, posT_hbm, d2_hbm, pos_v, ridx_v, cidx_v, d2_v):
    c = lax.axis_index("c")
    s = lax.axis_index("s")
    base = (c * _NS + s) * _EPT
    pltpu.sync_copy(posT_hbm, pos_v)
    pltpu.sync_copy(row_hbm.at[pl.ds(base, _EPT)], ridx_v)
    pltpu.sync_copy(col_hbm.at[pl.ds(base, _EPT)], cidx_v)

    @pl.loop(0, _EPT // _L)
    def _(g):
        sl = pl.ds(g * _L, _L)
        r = ridx_v[sl]
        cc = cidx_v[sl]
        acc = jnp.zeros((_L,), jnp.float32)
        for dim in range(3):
            di = jnp.full((_L,), dim, jnp.int32)
            pr = plsc.load_gather(pos_v, [di, r])
            pc = plsc.load_gather(pos_v, [di, cc])
            dd = pr - pc
            acc = acc + dd * dd
        d2_v[sl] = acc

    pltpu.sync_copy(d2_v, d2_hbm.at[pl.ds(base, _EPT)])


_d2_kernel = pl.kernel(
    _d2_body,
    out_type=jax.ShapeDtypeStruct((_E,), jnp.float32),
    mesh=_mesh,
    scratch_types=[
        pltpu.VMEM((3, _N), jnp.float32),
        pltpu.VMEM((_EPT,), jnp.int32),
        pltpu.VMEM((_EPT,), jnp.int32),
        pltpu.VMEM((_EPT,), jnp.float32),
    ],
    compiler_params=_sc_params,
)


def _aggr_body(h_hbm, ef_hbm, row_hbm, col_hbm, z_hbm, out_hbm,
               ridx_v, cidx_v, rows_v, ef_v, acc_sh):
    c = lax.axis_index("c")
    s = lax.axis_index("s")

    @pl.when(s < 15)
    def _():
        rsl = pl.ds(s * _RSTRIPE, _RSTRIPE)
        pltpu.sync_copy(z_hbm.at[rsl], acc_sh.at[rsl])

    @pl.when(s == 15)
    def _():
        rsl = pl.ds(15 * _RSTRIPE, _RLAST)
        pltpu.sync_copy(z_hbm.at[rsl], acc_sh.at[rsl])

    plsc.subcore_barrier()

    base0 = (c * _NS + s) * _EPT

    @pl.loop(0, _NCHUNK)
    def _(k):
        base = base0 + k * _KC
        pltpu.sync_copy(row_hbm.at[pl.ds(base, _KC)], ridx_v)
        pltpu.sync_copy(col_hbm.at[pl.ds(base, _KC)], cidx_v)
        pltpu.sync_copy(h_hbm.at[ridx_v], rows_v)
        pltpu.sync_copy(ef_hbm.at[pl.ds(base, _KC), :], ef_v)

        @pl.loop(0, _KC)
        def _(e):
            for j in range(_D // _L):
                sl = (pl.ds(e, 1), pl.ds(j * _L, _L))
                rows_v.at[*sl][...] = rows_v.at[*sl][...] * ef_v.at[*sl][...]

        pltpu.sync_copy(rows_v, acc_sh.at[cidx_v], add=True)

    plsc.subcore_barrier()

    @pl.when(s < 15)
    def _():
        rsl = pl.ds(s * _RSTRIPE, _RSTRIPE)
        pltpu.sync_copy(acc_sh.at[rsl], out_hbm.at[c, rsl])

    @pl.when(s == 15)
    def _():
        rsl = pl.ds(15 * _RSTRIPE, _RLAST)
        pltpu.sync_copy(acc_sh.at[rsl], out_hbm.at[c, rsl])


_aggr_kernel = pl.kernel(
    _aggr_body,
    out_type=jax.ShapeDtypeStruct((_NC, _N, _D), jnp.float32),
    mesh=_mesh,
    scratch_types=[
        pltpu.VMEM((_KC,), jnp.int32),
        pltpu.VMEM((_KC,), jnp.int32),
        pltpu.VMEM((_KC, _D), jnp.float32),
        pltpu.VMEM((_KC, _D), jnp.float32),
        pltpu.VMEM_SHARED((_N, _D), jnp.float32),
    ],
)


_BM = 2000
_BE = 3200


def _mm_body(x_ref, w_ref, b_ref, o_ref):
    o_ref[...] = (
        jnp.dot(x_ref[...], w_ref[...], preferred_element_type=jnp.float32)
        + b_ref[...]
    )


def _mm(x, W, b):
    return pl.pallas_call(
        _mm_body,
        grid=(_N // _BM,),
        in_specs=[
            pl.BlockSpec((_BM, _D), lambda i: (i, 0)),
            pl.BlockSpec((_D, _D), lambda i: (0, 0)),
            pl.BlockSpec((1, _D), lambda i: (0, 0)),
        ],
        out_specs=pl.BlockSpec((_BM, _D), lambda i: (i, 0)),
        out_shape=jax.ShapeDtypeStruct((_N, _D), jnp.float32),
    )(x, W, b.reshape(1, _D))


def _mm2_body(a0_ref, a1_ref, w_ref, b_ref, o_ref):
    a = a0_ref[...] + a1_ref[...]
    h = jnp.dot(a, w_ref[...], preferred_element_type=jnp.float32) + b_ref[...]
    o_ref[...] = jnp.maximum(h, 0.0)


def _mm2_relu(a0, a1, W, b):
    return pl.pallas_call(
        _mm2_body,
        grid=(_N // _BM,),
        in_specs=[
            pl.BlockSpec((_BM, _D), lambda i: (i, 0)),
            pl.BlockSpec((_BM, _D), lambda i: (i, 0)),
            pl.BlockSpec((_D, _D), lambda i: (0, 0)),
            pl.BlockSpec((1, _D), lambda i: (0, 0)),
        ],
        out_specs=pl.BlockSpec((_BM, _D), lambda i: (i, 0)),
        out_shape=jax.ShapeDtypeStruct((_N, _D), jnp.float32),
    )(a0, a1, W, b.reshape(1, _D))


def _ef_body(d2_ref, mu_ref, we_ref, be_ref, o_ref):
    dist = jnp.sqrt(d2_ref[...] + 1e-12)
    rbf = jnp.exp(-_GAMMA * (dist - mu_ref[...]) ** 2)
    o_ref[...] = (
        jnp.dot(rbf, we_ref[...], preferred_element_type=jnp.float32)
        + be_ref[...]
    )


def _ef(d2r, mu2, We, be):
    return pl.pallas_call(
        _ef_body,
        grid=(_E // _BE,),
        in_specs=[
            pl.BlockSpec((_BE, 1), lambda i: (i, 0)),
            pl.BlockSpec((1, _NRBF), lambda i: (0, 0)),
            pl.BlockSpec((_NRBF, _D), lambda i: (0, 0)),
            pl.BlockSpec((1, _D), lambda i: (0, 0)),
        ],
        out_specs=pl.BlockSpec((_BE, _D), lambda i: (i, 0)),
        out_shape=jax.ShapeDtypeStruct((_E, _D), jnp.float32),
    )(d2r, mu2, We, be.reshape(1, _D))


def _pool_body(bt_ref, h_ref, o_ref, cnt_ref):
    i = pl.program_id(0)

    @pl.when(i == 0)
    def _():
        o_ref[...] = jnp.zeros_like(o_ref)
        cnt_ref[...] = jnp.zeros_like(cnt_ref)

    b = bt_ref[...]
    gids = lax.broadcasted_iota(jnp.int32, (1, _NG), 1).astype(jnp.float32)
    mask = jnp.where(b == gids, 1.0, 0.0)
    dn = (((0,), (0,)), ((), ()))
    o_ref[...] += lax.dot_general(mask, h_ref[...], dn,
                                  preferred_element_type=jnp.float32)
    ones = jnp.ones_like(h_ref[...])
    cnt_ref[...] += lax.dot_general(mask, ones, dn,
                                    preferred_element_type=jnp.float32)

    @pl.when(i == pl.num_programs(0) - 1)
    def _():
        o_ref[...] = o_ref[...] / jnp.maximum(cnt_ref[...], 1.0)


def _pool(batch_f, h):
    return pl.pallas_call(
        _pool_body,
        grid=(_N // _BM,),
        in_specs=[
            pl.BlockSpec((_BM, 1), lambda i: (i, 0)),
            pl.BlockSpec((_BM, _D), lambda i: (i, 0)),
        ],
        out_specs=pl.BlockSpec((_NG, _D), lambda i: (0, 0)),
        out_shape=jax.ShapeDtypeStruct((_NG, _D), jnp.float32),
        scratch_shapes=[pltpu.VMEM((_NG, _D), jnp.float32)],
    )(batch_f, h)


def kernel(x, pos, edge_index, edge_attr, batch, W_in, b_in, mu, We1, be1, W1,
           b1, We2, be2, W2, b2, We3, be3, W3, b3):
    del edge_attr
    posT = pos.T
    row = edge_index[0]
    col = edge_index[1]
    d2 = _d2_kernel(row, col, posT)
    d2r = d2.reshape(_E, 1)
    mu2 = mu.reshape(1, _NRBF)

    h = _mm(x, W_in, b_in)
    z = jnp.zeros((_N, _D), jnp.float32)
    for We, be, W, b in ((We1, be1, W1, b1), (We2, be2, W2, b2),
                         (We3, be3, W3, b3)):
        ef = _ef(d2r, mu2, We, be)
        parts = _aggr_kernel(h, ef, row, col, z)
        h = _mm2_relu(parts[0], parts[1], W, b)

    return _pool(batch.astype(jnp.float32).reshape(_N, 1), h)


# speedup vs baseline: 5.5160x; 1.7493x over previous
import dataclasses
import functools

import jax
import jax.numpy as jnp
from jax import lax
from jax.experimental import pallas as pl
from jax.experimental.pallas import tpu as pltpu
from jax.experimental.pallas import tpu_sc as plsc

_N = 10000
_E = 320000
_D = 128
_NRBF = 32
_NG = 32
_GAMMA = 10.0

_NC = 2
_NS = 16
_NTILES = _NC * _NS
_EPT = _E // _NTILES
_KC = 40
_NCHUNK = _EPT // _KC
_RSTRIPE = 632
_RLAST = _N - 15 * _RSTRIPE
_L = 16

_mesh = plsc.VectorSubcoreMesh(core_axis_name="c", subcore_axis_name="s")

_sc_params = pltpu.CompilerParams()
if "needs_layout_passes" in pltpu.CompilerParams.__dataclass_fields__:
    _sc_params = dataclasses.replace(_sc_params, needs_layout_passes=False)


def _d2_body(row_hbm, col_hbm, posT_hbm, d2_hbm, pos_v, ridx_v, cidx_v, d2_v):
    c = lax.axis_index("c")
    s = lax.axis_index("s")
    base = (c * _NS + s) * _EPT
    pltpu.sync_copy(posT_hbm, pos_v)
    pltpu.sync_copy(row_hbm.at[pl.ds(base, _EPT)], ridx_v)
    pltpu.sync_copy(col_hbm.at[pl.ds(base, _EPT)], cidx_v)

    @pl.loop(0, _EPT // _L)
    def _(g):
        sl = pl.ds(g * _L, _L)
        r = ridx_v[sl]
        cc = cidx_v[sl]
        acc = jnp.zeros((_L,), jnp.float32)
        for dim in range(3):
            di = jnp.full((_L,), dim, jnp.int32)
            pr = plsc.load_gather(pos_v, [di, r])
            pc = plsc.load_gather(pos_v, [di, cc])
            dd = pr - pc
            acc = acc + dd * dd
        d2_v[sl] = acc

    pltpu.sync_copy(d2_v, d2_hbm.at[pl.ds(base, _EPT)])


_d2_kernel = pl.kernel(
    _d2_body,
    out_type=jax.ShapeDtypeStruct((_E,), jnp.float32),
    mesh=_mesh,
    scratch_types=[
        pltpu.VMEM((3, _N), jnp.float32),
        pltpu.VMEM((_EPT,), jnp.int32),
        pltpu.VMEM((_EPT,), jnp.int32),
        pltpu.VMEM((_EPT,), jnp.float32),
    ],
    compiler_params=_sc_params,
)


def _aggr_body(h_hbm, ef_hbm, row3_hbm, col3_hbm, z_hbm, out_hbm,
               rb0, rb1, cb0, cb1, rows0, rows1, ef0, ef1, acc_sh,
               sr0, sr1, sc0, sc1, sg0, sg1, se0, se1, ss0, ss1):
    c = lax.axis_index("c")
    s = lax.axis_index("s")
    w = c * _NS + s
    rbuf = (rb0, rb1)
    cbuf = (cb0, cb1)
    rows = (rows0, rows1)
    efs = (ef0, ef1)
    sr = (sr0, sr1)
    sc = (sc0, sc1)
    sg = (sg0, sg1)
    se = (se0, se1)
    ss = (ss0, ss1)
    gbase = w * _EPT

    @pl.when(s < 15)
    def _():
        rsl = pl.ds(s * _RSTRIPE, _RSTRIPE)
        pltpu.sync_copy(z_hbm.at[rsl], acc_sh.at[rsl])

    @pl.when(s == 15)
    def _():
        rsl = pl.ds(15 * _RSTRIPE, _RLAST)
        pltpu.sync_copy(z_hbm.at[rsl], acc_sh.at[rsl])

    plsc.subcore_barrier()

    def issue_ridx(j, b):
        pltpu.async_copy(row3_hbm.at[w, j], rbuf[b], sr[b])

    def wait_ridx(j, b):
        pltpu.make_async_copy(row3_hbm.at[w, j], rbuf[b], sr[b]).wait()

    def issue_cidx(j, b):
        pltpu.async_copy(col3_hbm.at[w, j], cbuf[b], sc[b])

    def wait_cidx(j, b):
        pltpu.make_async_copy(col3_hbm.at[w, j], cbuf[b], sc[b]).wait()

    def issue_ge(j, b):
        pltpu.async_copy(h_hbm.at[rbuf[b]], rows[b], sg[b])
        pltpu.async_copy(ef_hbm.at[pl.ds(gbase + j * _KC, _KC), :], efs[b], se[b])

    def wait_ge(j, b):
        pltpu.make_async_copy(h_hbm.at[rbuf[b]], rows[b], sg[b]).wait()
        pltpu.make_async_copy(
            ef_hbm.at[pl.ds(gbase + j * _KC, _KC), :], efs[b], se[b]).wait()

    def scat(j, b):
        pltpu.async_copy(rows[b], acc_sh.at[cbuf[b]], ss[b], add=True)

    def wait_scat(j, b):
        pltpu.make_async_copy(rows[b], acc_sh.at[cbuf[b]], ss[b]).wait()

    def mult(b):
        @pl.loop(0, _KC, step=8)
        def _(e):
            for ee in range(8):
                for j in range(_D // _L):
                    sl = (pl.ds(e + ee, 1), pl.ds(j * _L, _L))
                    rows[b].at[*sl][...] = rows[b].at[*sl][...] * efs[b].at[*sl][...]

    def chunk_body(k, b, drain_prev=True, issue_next=True, issue_r2=True,
                   issue_c1=True):
        if drain_prev:
            wait_scat(k - 1, 1 - b)
        if drain_prev and issue_c1:
            issue_cidx(k + 1, 1 - b)
        if issue_next:
            wait_ridx(k + 1, 1 - b)
            issue_ge(k + 1, 1 - b)
        wait_ge(k, b)
        if issue_r2:
            issue_ridx(k + 2, b)
        mult(b)
        wait_cidx(k, b)
        scat(k, b)

    issue_ridx(0, 0)
    issue_ridx(1, 1)
    issue_cidx(0, 0)
    issue_cidx(1, 1)
    wait_ridx(0, 0)
    issue_ge(0, 0)
    chunk_body(0, 0, drain_prev=False)

    @pl.loop(0, (_NCHUNK - 4) // 2)
    def _(t):
        chunk_body(2 * t + 1, 1)
        chunk_body(2 * t + 2, 0)

    chunk_body(_NCHUNK - 3, 1)
    chunk_body(_NCHUNK - 2, 0, issue_r2=False)
    chunk_body(_NCHUNK - 1, 1, issue_next=False, issue_r2=False,
               issue_c1=False)
    wait_scat(_NCHUNK - 1, 1)

    plsc.subcore_barrier()

    @pl.when(s < 15)
    def _():
        rsl = pl.ds(s * _RSTRIPE, _RSTRIPE)
        pltpu.sync_copy(acc_sh.at[rsl], out_hbm.at[c, rsl])

    @pl.when(s == 15)
    def _():
        rsl = pl.ds(15 * _RSTRIPE, _RLAST)
        pltpu.sync_copy(acc_sh.at[rsl], out_hbm.at[c, rsl])


_aggr_kernel = pl.kernel(
    _aggr_body,
    out_type=jax.ShapeDtypeStruct((_NC, _N, _D), jnp.float32),
    mesh=_mesh,
    scratch_types=[
        pltpu.VMEM((_KC,), jnp.int32),
        pltpu.VMEM((_KC,), jnp.int32),
        pltpu.VMEM((_KC,), jnp.int32),
        pltpu.VMEM((_KC,), jnp.int32),
        pltpu.VMEM((_KC, _D), jnp.float32),
        pltpu.VMEM((_KC, _D), jnp.float32),
        pltpu.VMEM((_KC, _D), jnp.float32),
        pltpu.VMEM((_KC, _D), jnp.float32),
        pltpu.VMEM_SHARED((_N, _D), jnp.float32),
    ] + [pltpu.SemaphoreType.DMA] * 10,
)


_BM = 2000
_BE = 3200


def _mm_body(x_ref, w_ref, b_ref, o_ref):
    o_ref[...] = (
        jnp.dot(x_ref[...], w_ref[...], preferred_element_type=jnp.float32)
        + b_ref[...]
    )


def _mm(x, W, b):
    return pl.pallas_call(
        _mm_body,
        grid=(_N // _BM,),
        in_specs=[
            pl.BlockSpec((_BM, _D), lambda i: (i, 0)),
            pl.BlockSpec((_D, _D), lambda i: (0, 0)),
            pl.BlockSpec((1, _D), lambda i: (0, 0)),
        ],
        out_specs=pl.BlockSpec((_BM, _D), lambda i: (i, 0)),
        out_shape=jax.ShapeDtypeStruct((_N, _D), jnp.float32),
    )(x, W, b.reshape(1, _D))


def _mm2_body(a0_ref, a1_ref, w_ref, b_ref, o_ref):
    a = a0_ref[...] + a1_ref[...]
    h = jnp.dot(a, w_ref[...], preferred_element_type=jnp.float32) + b_ref[...]
    o_ref[...] = jnp.maximum(h, 0.0)


def _mm2_relu(a0, a1, W, b):
    return pl.pallas_call(
        _mm2_body,
        grid=(_N // _BM,),
        in_specs=[
            pl.BlockSpec((_BM, _D), lambda i: (i, 0)),
            pl.BlockSpec((_BM, _D), lambda i: (i, 0)),
            pl.BlockSpec((_D, _D), lambda i: (0, 0)),
            pl.BlockSpec((1, _D), lambda i: (0, 0)),
        ],
        out_specs=pl.BlockSpec((_BM, _D), lambda i: (i, 0)),
        out_shape=jax.ShapeDtypeStruct((_N, _D), jnp.float32),
    )(a0, a1, W, b.reshape(1, _D))


def _ef_body(d2_ref, mu_ref, we_ref, be_ref, o_ref):
    dist = jnp.sqrt(d2_ref[...] + 1e-12)
    rbf = jnp.exp(-_GAMMA * (dist - mu_ref[...]) ** 2)
    o_ref[...] = (
        jnp.dot(rbf, we_ref[...], preferred_element_type=jnp.float32)
        + be_ref[...]
    )


def _ef(d2r, mu2, We, be):
    return pl.pallas_call(
        _ef_body,
        grid=(_E // _BE,),
        in_specs=[
            pl.BlockSpec((_BE, 1), lambda i: (i, 0)),
            pl.BlockSpec((1, _NRBF), lambda i: (0, 0)),
            pl.BlockSpec((_NRBF, _D), lambda i: (0, 0)),
            pl.BlockSpec((1, _D), lambda i: (0, 0)),
        ],
        out_specs=pl.BlockSpec((_BE, _D), lambda i: (i, 0)),
        out_shape=jax.ShapeDtypeStruct((_E, _D), jnp.float32),
    )(d2r, mu2, We, be.reshape(1, _D))


def _pool_body(bt_ref, h_ref, o_ref, cnt_ref):
    i = pl.program_id(0)

    @pl.when(i == 0)
    def _():
        o_ref[...] = jnp.zeros_like(o_ref)
        cnt_ref[...] = jnp.zeros_like(cnt_ref)

    b = bt_ref[...]
    gids = lax.broadcasted_iota(jnp.int32, (1, _NG), 1).astype(jnp.float32)
    mask = jnp.where(b == gids, 1.0, 0.0)
    dn = (((0,), (0,)), ((), ()))
    o_ref[...] += lax.dot_general(mask, h_ref[...], dn,
                                  preferred_element_type=jnp.float32)
    ones = jnp.ones_like(h_ref[...])
    cnt_ref[...] += lax.dot_general(mask, ones, dn,
                                    preferred_element_type=jnp.float32)

    @pl.when(i == pl.num_programs(0) - 1)
    def _():
        o_ref[...] = o_ref[...] / jnp.maximum(cnt_ref[...], 1.0)


def _pool(batch_f, h):
    return pl.pallas_call(
        _pool_body,
        grid=(_N // _BM,),
        in_specs=[
            pl.BlockSpec((_BM, 1), lambda i: (i, 0)),
            pl.BlockSpec((_BM, _D), lambda i: (i, 0)),
        ],
        out_specs=pl.BlockSpec((_NG, _D), lambda i: (0, 0)),
        out_shape=jax.ShapeDtypeStruct((_NG, _D), jnp.float32),
        scratch_shapes=[pltpu.VMEM((_NG, _D), jnp.float32)],
    )(batch_f, h)


def kernel(x, pos, edge_index, edge_attr, batch, W_in, b_in, mu, We1, be1, W1,
           b1, We2, be2, W2, b2, We3, be3, W3, b3):
    del edge_attr
    posT = pos.T
    row = edge_index[0]
    col = edge_index[1]
    row3 = row.reshape(_NTILES, _NCHUNK, _KC)
    col3 = col.reshape(_NTILES, _NCHUNK, _KC)
    d2 = _d2_kernel(row, col, posT)
    d2r = d2.reshape(_E, 1)
    mu2 = mu.reshape(1, _NRBF)

    h = _mm(x, W_in, b_in)
    z = jnp.zeros((_N, _D), jnp.float32)
    for We, be, W, b in ((We1, be1, W1, b1), (We2, be2, W2, b2),
                         (We3, be3, W3, b3)):
        ef = _ef(d2r, mu2, We, be)
        parts = _aggr_kernel(h, ef, row3, col3, z)
        h = _mm2_relu(parts[0], parts[1], W, b)

    return _pool(batch.astype(jnp.float32).reshape(_N, 1), h)
